# baseline (speedup 1.0000x reference)
import functools

import jax
import jax.numpy as jnp
from jax import lax
from jax.experimental import pallas as pl
from jax.experimental.pallas import tpu as pltpu
from jax.experimental.pallas import tpu_sc as plsc

_B = 8192
_K = 8192
_CODE_DIM = 32
_H = 256
_TWO_V = 128
_CW = _CODE_DIM + _TWO_V

_TB = 256
_KC = 2048


def _main_body(x_ref, w1_ref, b1_ref, w2_ref, b2_ref, cb_ref,
               dw1_ref, db1_ref, dw2_ref, db2_ref,
               ze_ref, idx_ref, comb_ref, cnorm_ref):
    pid = pl.program_id(0)

    @pl.when(pid == 0)
    def _init_cnorm():
        cbf = cb_ref[...]
        cnorm_ref[...] = jnp.sum(cbf * cbf, axis=-1).reshape(1, _K)

    xb = x_ref[...]
    h = jnp.maximum(jnp.dot(xb, w1_ref[...],
                            preferred_element_type=jnp.float32)
                    + b1_ref[...], 0.0)
    z = (jnp.dot(h, w2_ref[...], preferred_element_type=jnp.float32)
         + b2_ref[...])
    ze_ref[...] = z

    ct = cb_ref[pl.ds(pid * _TB, _TB), :]
    h2 = jnp.maximum(jnp.dot(ct, dw1_ref[...],
                             preferred_element_type=jnp.float32)
                     + db1_ref[...], 0.0)
    lg = (jnp.dot(h2, dw2_ref[...], preferred_element_type=jnp.float32)
          + db2_ref[...])
    comb_ref[...] = jnp.concatenate([ct, lg], axis=1)

    znorm = jnp.sum(z ** 2, axis=-1, keepdims=True)
    iota = lax.broadcasted_iota(jnp.int32, (_TB, _KC), 1)
    gmin = jnp.full((_TB, 1), jnp.inf, dtype=jnp.float32)
    gidx = jnp.zeros((_TB, 1), dtype=jnp.int32)
    for c in range(_K // _KC):
        cb = cb_ref[pl.ds(c * _KC, _KC), :]
        zc2 = lax.dot_general(z, cb + cb, (((1,), (1,)), ((), ())),
                              preferred_element_type=jnp.float32)
        dist = znorm - zc2 + cnorm_ref[0:1, pl.ds(c * _KC, _KC)]
        cmin = jnp.min(dist, axis=1, keepdims=True)
        cidx = jnp.min(jnp.where(dist == cmin, iota, _K),
                       axis=1, keepdims=True) + c * _KC
        take = cmin < gmin
        gidx = jnp.where(take, cidx, gidx)
        gmin = jnp.where(take, cmin, gmin)
    idx_ref[...] = gidx.reshape(1, 1, _TB)


def _main_call(xf, enc_w1, enc_b1, enc_w2, enc_b2, codebook,
               dec_w1, dec_b1, dec_w2, dec_b2):
    nb = _B // _TB
    return pl.pallas_call(
        _main_body,
        grid=(nb,),
        in_specs=[
            pl.BlockSpec((_TB, _TWO_V), lambda i: (i, 0)),
            pl.BlockSpec((_TWO_V, _H), lambda i: (0, 0)),
            pl.BlockSpec((1, _H), lambda i: (0, 0)),
            pl.BlockSpec((_H, _CODE_DIM), lambda i: (0, 0)),
            pl.BlockSpec((1, _CODE_DIM), lambda i: (0, 0)),
            pl.BlockSpec((_K, _CODE_DIM), lambda i: (0, 0)),
            pl.BlockSpec((_CODE_DIM, _H), lambda i: (0, 0)),
            pl.BlockSpec((1, _H), lambda i: (0, 0)),
            pl.BlockSpec((_H, _TWO_V), lambda i: (0, 0)),
            pl.BlockSpec((1, _TWO_V), lambda i: (0, 0)),
        ],
        out_specs=[
            pl.BlockSpec((_TB, _CODE_DIM), lambda i: (i, 0)),
            pl.BlockSpec((1, 1, _TB), lambda i: (i, 0, 0)),
            pl.BlockSpec((_TB, _CW), lambda i: (i, 0)),
        ],
        out_shape=[
            jax.ShapeDtypeStruct((_B, _CODE_DIM), jnp.float32),
            jax.ShapeDtypeStruct((nb, 1, _TB), jnp.int32),
            jax.ShapeDtypeStruct((_K, _CW), jnp.float32),
        ],
        scratch_shapes=[pltpu.VMEM((1, _K), jnp.float32)],
    )(xf, enc_w1, enc_b1.reshape(1, _H), enc_w2, enc_b2.reshape(1, _CODE_DIM),
      codebook, dec_w1, dec_b1.reshape(1, _H), dec_w2,
      dec_b2.reshape(1, _TWO_V))


@functools.cache
def _make_sc_gather():
    info = plsc.get_sparse_core_info()
    nc, ns = info.num_cores, info.num_subcores
    nw = nc * ns
    bw = _B // nw
    mesh = plsc.VectorSubcoreMesh(core_axis_name="c", subcore_axis_name="s")

    @functools.partial(
        pl.kernel, mesh=mesh,
        out_type=jax.ShapeDtypeStruct((_B, _CW), jnp.float32),
        compiler_params=pltpu.CompilerParams(use_tc_tiling_on_sc=False),
        scratch_types=[
            pltpu.VMEM((bw,), jnp.int32),
            pltpu.VMEM((bw, _CW), jnp.float32),
            pltpu.SemaphoreType.DMA,
        ],
    )
    def gather(tab_hbm, idx_hbm, out_hbm, idx_v, rows_v, sem):
        wid = lax.axis_index("s") * nc + lax.axis_index("c")
        base = wid * bw
        pltpu.sync_copy(idx_hbm.at[pl.ds(base, bw)], idx_v)
        pltpu.async_copy(tab_hbm.at[idx_v], rows_v, sem).wait()
        pltpu.sync_copy(rows_v, out_hbm.at[pl.ds(base, bw)])

    return gather


def kernel(x, enc_w1, enc_b1, enc_w2, enc_b2, dec_w1, dec_b1, dec_w2, dec_b2,
           codebook):
    b = x.shape[0]
    xf = x.reshape(b, -1)
    z_e_x, idx3, comb = _main_call(xf, enc_w1, enc_b1, enc_w2, enc_b2,
                                   codebook, dec_w1, dec_b1, dec_w2, dec_b2)
    x_code_idx = idx3.reshape(b)
    g = _make_sc_gather()(comb, x_code_idx)
    z_q_x = g[:, :_CODE_DIM]
    logits = g[:, _CODE_DIM:]
    return (logits.reshape(b, 2, _TWO_V // 2), z_e_x, z_q_x, x_code_idx)


# speedup vs baseline: 1.3231x; 1.3231x over previous
import functools

import jax
import jax.numpy as jnp
from jax import lax
from jax.experimental import pallas as pl
from jax.experimental.pallas import tpu as pltpu
from jax.experimental.pallas import tpu_sc as plsc

_B = 8192
_K = 8192
_CODE_DIM = 32
_H = 256
_TWO_V = 128

_TB = 256
_KC = 2048
_G = 128


def _enc_argmin_body(x_ref, w1_ref, b1_ref, w2_ref, b2_ref, cb_ref,
                     ze_ref, idx_ref, cnorm_ref):
    @pl.when(pl.program_id(0) == 0)
    def _init_cnorm():
        cbf = cb_ref[...]
        cnorm_ref[...] = lax.dot_general(
            jnp.ones((1, _CODE_DIM), jnp.float32), cbf * cbf,
            (((1,), (1,)), ((), ())), preferred_element_type=jnp.float32)

    xb = x_ref[...]
    h = jnp.maximum(jnp.dot(xb, w1_ref[...],
                            preferred_element_type=jnp.float32)
                    + b1_ref[...], 0.0)
    z = (jnp.dot(h, w2_ref[...], preferred_element_type=jnp.float32)
         + b2_ref[...])
    ze_ref[...] = z

    znorm = jnp.sum(z ** 2, axis=-1, keepdims=True)

    v = jnp.full((_TB, _G), jnp.inf, dtype=jnp.float32)
    ibase = jnp.zeros((_TB, _G), dtype=jnp.int32)
    for c in range(_K // _KC):
        cb = cb_ref[pl.ds(c * _KC, _KC), :]
        zc2 = lax.dot_general(z, cb + cb, (((1,), (1,)), ((), ())),
                              preferred_element_type=jnp.float32)
        for g in range(_KC // _G):
            base = c * _KC + g * _G
            dg = (znorm - zc2[:, g * _G:(g + 1) * _G]
                  + cnorm_ref[0:1, pl.ds(base, _G)])
            take = dg < v
            v = jnp.where(take, dg, v)
            ibase = jnp.where(take, base, ibase)
    ielem = ibase + lax.broadcasted_iota(jnp.int32, (_TB, _G), 1)
    cmin = jnp.min(v, axis=1, keepdims=True)
    gidx = jnp.min(jnp.where(v == cmin, ielem, _K),
                   axis=1, keepdims=True)
    idx_ref[...] = gidx.reshape(1, 1, _TB)


def _enc_argmin(xf, enc_w1, enc_b1, enc_w2, enc_b2, codebook):
    nb = _B // _TB
    return pl.pallas_call(
        _enc_argmin_body,
        grid=(nb,),
        in_specs=[
            pl.BlockSpec((_TB, _TWO_V), lambda i: (i, 0)),
            pl.BlockSpec((_TWO_V, _H), lambda i: (0, 0)),
            pl.BlockSpec((1, _H), lambda i: (0, 0)),
            pl.BlockSpec((_H, _CODE_DIM), lambda i: (0, 0)),
            pl.BlockSpec((1, _CODE_DIM), lambda i: (0, 0)),
            pl.BlockSpec((_K, _CODE_DIM), lambda i: (0, 0)),
        ],
        out_specs=[
            pl.BlockSpec((_TB, _CODE_DIM), lambda i: (i, 0)),
            pl.BlockSpec((1, 1, _TB), lambda i: (i, 0, 0)),
        ],
        out_shape=[
            jax.ShapeDtypeStruct((_B, _CODE_DIM), jnp.float32),
            jax.ShapeDtypeStruct((nb, 1, _TB), jnp.int32),
        ],
        scratch_shapes=[pltpu.VMEM((1, _K), jnp.float32)],
    )(xf, enc_w1, enc_b1.reshape(1, _H), enc_w2, enc_b2.reshape(1, _CODE_DIM),
      codebook)


@functools.cache
def _make_sc_gather():
    info = plsc.get_sparse_core_info()
    nc, ns = info.num_cores, info.num_subcores
    nw = nc * ns
    bw = _B // nw
    mesh = plsc.VectorSubcoreMesh(core_axis_name="c", subcore_axis_name="s")

    @functools.partial(
        pl.kernel, mesh=mesh,
        out_type=jax.ShapeDtypeStruct((_B, _CODE_DIM), jnp.float32),
        compiler_params=pltpu.CompilerParams(use_tc_tiling_on_sc=False),
        scratch_types=[
            pltpu.VMEM((bw,), jnp.int32),
            pltpu.VMEM((bw, _CODE_DIM), jnp.float32),
            pltpu.SemaphoreType.DMA,
        ],
    )
    def gather(table_hbm, idx_hbm, out_hbm, idx_v, rows_v, sem):
        wid = lax.axis_index("s") * nc + lax.axis_index("c")
        base = wid * bw
        pltpu.sync_copy(idx_hbm.at[pl.ds(base, bw)], idx_v)
        pltpu.async_copy(table_hbm.at[idx_v], rows_v, sem).wait()
        pltpu.sync_copy(rows_v, out_hbm.at[pl.ds(base, bw)])

    return gather


def _decoder_body(zq_ref, w1_ref, b1_ref, w2_ref, b2_ref, out_ref):
    h2 = jnp.maximum(jnp.dot(zq_ref[...], w1_ref[...],
                             preferred_element_type=jnp.float32)
                     + b1_ref[...], 0.0)
    out_ref[...] = (jnp.dot(h2, w2_ref[...],
                            preferred_element_type=jnp.float32)
                    + b2_ref[...])


def _decoder(zq, dec_w1, dec_b1, dec_w2, dec_b2):
    td = 1024
    return pl.pallas_call(
        _decoder_body,
        grid=(_B // td,),
        in_specs=[
            pl.BlockSpec((td, _CODE_DIM), lambda i: (i, 0)),
            pl.BlockSpec((_CODE_DIM, _H), lambda i: (0, 0)),
            pl.BlockSpec((1, _H), lambda i: (0, 0)),
            pl.BlockSpec((_H, _TWO_V), lambda i: (0, 0)),
            pl.BlockSpec((1, _TWO_V), lambda i: (0, 0)),
        ],
        out_specs=pl.BlockSpec((td, _TWO_V), lambda i: (i, 0)),
        out_shape=jax.ShapeDtypeStruct((_B, _TWO_V), jnp.float32),
    )(zq, dec_w1, dec_b1.reshape(1, _H), dec_w2, dec_b2.reshape(1, _TWO_V))


def kernel(x, enc_w1, enc_b1, enc_w2, enc_b2, dec_w1, dec_b1, dec_w2, dec_b2,
           codebook):
    b = x.shape[0]
    xf = x.reshape(b, -1)
    z_e_x, idx3 = _enc_argmin(xf, enc_w1, enc_b1, enc_w2, enc_b2, codebook)
    x_code_idx = idx3.reshape(b)
    z_q_x = _make_sc_gather()(codebook, x_code_idx)
    logits = _decoder(z_q_x, dec_w1, dec_b1, dec_w2, dec_b2)
    return (logits.reshape(b, 2, _TWO_V // 2), z_e_x, z_q_x, x_code_idx)
